# baseline (speedup 1.0000x reference)
import jax
import jax.numpy as jnp
from jax.experimental import pallas as pl


def _gat_jnp(x, edge_index, W, a_src, a_dst, b):
    n = x.shape[0]
    h = x @ W.T
    src = edge_index[0]
    dst = edge_index[1]
    s = jnp.sum(h * a_src, axis=-1)
    t = jnp.sum(h * a_dst, axis=-1)
    e = jax.nn.leaky_relu(s[src] + t[dst], 0.2)
    emax = jax.ops.segment_max(e, dst, num_segments=n)
    emax = jnp.where(jnp.isfinite(emax), emax, 0.0)
    ee = jnp.exp(e - emax[dst])
    denom = jax.ops.segment_sum(ee, dst, num_segments=n)
    alpha = ee / (denom[dst] + 1e-16)
    return jax.ops.segment_sum(h[src] * alpha[:, None], dst, num_segments=n) + b


def _identity_pallas(x):
    def body(x_ref, o_ref):
        o_ref[...] = x_ref[...]
    return pl.pallas_call(
        body, out_shape=jax.ShapeDtypeStruct(x.shape, x.dtype))(x)


def kernel(x, edge_index, batch, epoch, params):
    dst = edge_index[1].astype(jnp.int32)
    order = jnp.argsort(dst)
    srcs = edge_index[0].astype(jnp.int32)[order]
    dsts = dst[order]
    rowptr = jnp.searchsorted(dsts, jnp.arange(x.shape[0] + 1, dtype=jnp.int32))
    x = x + 0.0 * (jnp.float32(srcs[0] + rowptr[0]))

    x = _identity_pallas(x)
    x = x / jnp.maximum(jnp.sum(jnp.abs(x), axis=-1, keepdims=True), 1e-12)
    h = x
    for i in range(1, 5):
        h = jnp.tanh(_gat_jnp(h, edge_index, params['Wc%d' % i], params['asrc%d' % i], params['adst%d' % i], params['bc%d' % i]))
    latent = h
    m = latent
    for i in range(1, 4):
        m = jnp.tanh(m @ params['Wm%d' % i].T + params['bm%d' % i])
    m4 = m @ params['Wm4'].T + params['bm4']
    sm = jax.nn.softmax(m4, axis=1)
    pos = jnp.arange(2, dtype=sm.dtype)
    soft_argmax = jnp.sum(sm * pos, axis=1, keepdims=True)
    d = latent * soft_argmax
    for i in range(5, 9):
        d = jnp.tanh(_gat_jnp(d, edge_index, params['Wc%d' % i], params['asrc%d' % i], params['adst%d' % i], params['bc%d' % i]))
    return d


# speedup vs baseline: 10.9433x; 10.9433x over previous
import dataclasses
import functools

import jax
import jax.numpy as jnp
from jax import lax
from jax.experimental import pallas as pl
from jax.experimental.pallas import tpu as pltpu
from jax.experimental.pallas import tpu_sc as plsc

N_RAW = 50000
E_RAW = 800000
N_PAD = 50176
E_PAD = E_RAW + 64
RP_LEN = N_PAD + 128
NB_LEN = 48
CE = 64
NS = 64
R = 512
EPS = 1e-16

_f32 = jnp.float32
_i32 = jnp.int32


def _vsplat(v16, j):
    idx = jnp.full((16,), j, _i32)
    return v16.at[idx].get(mode="promise_in_bounds")


def _make_sc_gat(FC):
    F = FC * 16
    mesh = plsc.VectorSubcoreMesh(core_axis_name="c", subcore_axis_name="s")

    cp = pltpu.CompilerParams()
    if "needs_layout_passes" in pltpu.CompilerParams.__dataclass_fields__:
        cp = dataclasses.replace(cp, needs_layout_passes=False)

    @functools.partial(
        pl.kernel,
        mesh=mesh,
        compiler_params=cp,
        out_type=jax.ShapeDtypeStruct((N_PAD * F,), _f32),
        scratch_types=[
            pltpu.VMEM((N_PAD,), _f32),
            pltpu.VMEM((NS,), _f32),
            pltpu.VMEM((CE,), _i32),
            pltpu.VMEM((CE,), _i32),
            pltpu.VMEM((CE, F), _f32),
            pltpu.VMEM((NS * F,), _f32),
            pltpu.VMEM((NS,), _f32),
            pltpu.VMEM((96,), _i32),
            pltpu.VMEM((NB_LEN,), _i32),
            pltpu.SemaphoreType.DMA,
        ],
    )
    def sc_gat(h_hbm, s_hbm, t_hbm, srcs_hbm, dsts_hbm, rp_hbm, nb_hbm,
               num_hbm,
               s_v, t_v, si_v, dl_v, hrows_v, acc_v, dac_v, rp_v, nb_v, sem):
        lane = lax.iota(_i32, 16)
        zero16 = jnp.zeros((16,), _f32)
        wid = lax.axis_index("s") * 2 + lax.axis_index("c")

        pltpu.sync_copy(nb_hbm, nb_v)
        pltpu.sync_copy(s_hbm, s_v)

        nb0 = nb_v[pl.ds(0, 16)]
        nb1 = nb_v[pl.ds(16, 16)]
        nb2 = nb_v[pl.ds(32, 16)]

        def nbsel(i):
            q = i // 16
            r = i - q * 16
            sel = jnp.where(q == 0, nb0, jnp.where(q == 1, nb1, nb2))
            return lax.reduce_max(jnp.where(lane == r, sel, 0), (0,))

        n_lo = nbsel(wid)
        n_hi = nbsel(wid + 1)
        nslices = (n_hi - n_lo) // NS

        def slice_body(sl, carry):
            m0 = pl.multiple_of(n_lo + sl * NS, 64)

            @pl.loop(0, NS * F, step=16)
            def _zn(i):
                acc_v[pl.ds(i, 16)] = zero16

            dac_v[pl.ds(0, 16)] = zero16
            dac_v[pl.ds(16, 16)] = zero16
            dac_v[pl.ds(32, 16)] = zero16
            dac_v[pl.ds(48, 16)] = zero16

            pltpu.sync_copy(t_hbm.at[pl.ds(m0, NS)], t_v)
            pltpu.sync_copy(rp_hbm.at[pl.ds(m0, 96)], rp_v)
            rpa = rp_v[pl.ds(0, 16)]
            rpb = rp_v[pl.ds(64, 16)]
            rp_lo = lax.reduce_max(jnp.where(lane == 0, rpa, 0), (0,))
            rp_hi = lax.reduce_max(jnp.where(lane == 0, rpb, 0), (0,))
            c0 = rp_lo // CE
            nch = (rp_hi + (CE - 1)) // CE - c0

            def chunk_body(ci, ccarry):
                base = pl.multiple_of((c0 + ci) * CE, 64)
                pltpu.sync_copy(srcs_hbm.at[pl.ds(base, CE)], si_v)
                pltpu.sync_copy(dsts_hbm.at[pl.ds(base, CE)], dl_v)
                pltpu.async_copy(h_hbm.at[si_v], hrows_v, sem).wait()

                ee_l = []
                dc_l = []
                for q in range(CE // 16):
                    si16 = si_v[pl.ds(q * 16, 16)]
                    sg = plsc.load_gather(s_v, [si16])
                    d16 = dl_v[pl.ds(q * 16, 16)] - m0
                    valid = (d16 >= 0) & (d16 < NS)
                    dc16 = jnp.clip(d16, 0, NS - 1)
                    tg = plsc.load_gather(t_v, [dc16])
                    z = sg + tg
                    zl = jnp.where(z > 0.0, z, 0.2 * z)
                    ee_l.append(jnp.where(valid, jnp.exp(zl), 0.0))
                    dc_l.append(dc16)

                mask0 = lane == 0
                for q in range(CE // 16):
                    for r_ in range(16):
                        j = q * 16 + r_
                        w16 = _vsplat(ee_l[q], r_)
                        d16s = _vsplat(dc_l[q], r_)
                        rowb = d16s * F
                        for k in range(FC):
                            idx = rowb + (lane + (k * 16))
                            vals = w16 * hrows_v[j, pl.ds(k * 16, 16)]
                            plsc.addupdate_scatter(acc_v, [idx], vals)
                        plsc.addupdate_scatter(dac_v, [d16s], w16, mask=mask0)
                return ccarry

            lax.fori_loop(0, nch, chunk_body, 0)

            for g in range(NS // 16):
                den16 = dac_v[pl.ds(g * 16, 16)]
                rec16 = 1.0 / (den16 + EPS)
                for r_ in range(16):
                    rr = g * 16 + r_
                    rec = _vsplat(rec16, r_)
                    for k in range(FC):
                        off = rr * F + k * 16
                        acc_v[pl.ds(off, 16)] = rec * acc_v[pl.ds(off, 16)]

            pltpu.sync_copy(acc_v, num_hbm.at[pl.ds(m0 * F, NS * F)])
            return carry

        lax.fori_loop(0, nslices, slice_body, 0)

    return sc_gat


_sc_gat_128 = _make_sc_gat(8)


def _st_out(h, asv, adv, s_ref, t_ref):
    s_ref[...] = jnp.sum(h * asv, axis=1).reshape(1, 4, 128)
    t_ref[...] = jnp.sum(h * adv, axis=1).reshape(1, 4, 128)


_ST_SPEC = pl.BlockSpec((1, 4, 128), lambda i: (i, 0, 0))
_ST_SHAPE = jax.ShapeDtypeStruct((N_PAD // R, 4, 128), _f32)


def _tc_first(x, wt, asv, adv):
    din = x.shape[1]

    def body(x_ref, w_ref, as_ref, ad_ref, h_ref, s_ref, t_ref):
        xb = x_ref[...]
        xin = xb / jnp.maximum(jnp.sum(jnp.abs(xb), axis=1, keepdims=True),
                               1e-12)
        h = jnp.dot(xin, w_ref[...], preferred_element_type=_f32)
        h_ref[...] = h
        _st_out(h, as_ref[...], ad_ref[...], s_ref, t_ref)

    return pl.pallas_call(
        body,
        grid=(N_PAD // R,),
        in_specs=[
            pl.BlockSpec((R, din), lambda i: (i, 0)),
            pl.BlockSpec((din, 128), lambda i: (0, 0)),
            pl.BlockSpec((1, 128), lambda i: (0, 0)),
            pl.BlockSpec((1, 128), lambda i: (0, 0)),
        ],
        out_specs=[
            pl.BlockSpec((R, 128), lambda i: (i, 0)),
            _ST_SPEC,
            _ST_SPEC,
        ],
        out_shape=[
            jax.ShapeDtypeStruct((N_PAD, 128), _f32),
            _ST_SHAPE,
            _ST_SHAPE,
        ],
    )(x, wt, asv, adv)


def _tc_mid(num, bprev, wt, asv, adv, dout):
    def body(n_ref, b_ref, w_ref, as_ref, ad_ref, h_ref, s_ref, t_ref):
        xin = jnp.tanh(n_ref[...] + b_ref[...])
        h = jnp.dot(xin, w_ref[...], preferred_element_type=_f32)
        h_ref[...] = h
        _st_out(h, as_ref[...], ad_ref[...], s_ref, t_ref)

    return pl.pallas_call(
        body,
        grid=(N_PAD // R,),
        in_specs=[
            pl.BlockSpec((R, 128), lambda i: (i, 0)),
            pl.BlockSpec((1, 128), lambda i: (0, 0)),
            pl.BlockSpec((128, dout), lambda i: (0, 0)),
            pl.BlockSpec((1, dout), lambda i: (0, 0)),
            pl.BlockSpec((1, dout), lambda i: (0, 0)),
        ],
        out_specs=[
            pl.BlockSpec((R, dout), lambda i: (i, 0)),
            _ST_SPEC,
            _ST_SPEC,
        ],
        out_shape=[
            jax.ShapeDtypeStruct((N_PAD, dout), _f32),
            _ST_SHAPE,
            _ST_SHAPE,
        ],
    )(num, bprev, wt, asv, adv)


def _tc_mask(num4, b4, wm1t, bm1, wm2t, bm2, wm3t, bm3, wm4d, bm4ds,
             wt5, asv, adv):
    def body(n_ref, b_ref, w1_ref, b1_ref, w2_ref, b2_ref, w3_ref, b3_ref,
             w4_ref, b4d_ref, w5_ref, as_ref, ad_ref, h_ref, s_ref, t_ref):
        latent = jnp.tanh(n_ref[...] + b_ref[...])
        m = jnp.tanh(jnp.dot(latent, w1_ref[...],
                             preferred_element_type=_f32) + b1_ref[...])
        m = jnp.tanh(jnp.dot(m, w2_ref[...],
                             preferred_element_type=_f32) + b2_ref[...])
        m = jnp.tanh(jnp.dot(m, w3_ref[...],
                             preferred_element_type=_f32) + b3_ref[...])
        logit = jnp.sum(m * w4_ref[...] + b4d_ref[...], axis=1,
                        keepdims=True)
        sa = jax.nn.sigmoid(logit)
        d0 = latent * sa
        h = jnp.dot(d0, w5_ref[...], preferred_element_type=_f32)
        h_ref[...] = h
        _st_out(h, as_ref[...], ad_ref[...], s_ref, t_ref)

    return pl.pallas_call(
        body,
        grid=(N_PAD // R,),
        in_specs=[
            pl.BlockSpec((R, 128), lambda i: (i, 0)),
            pl.BlockSpec((1, 128), lambda i: (0, 0)),
            pl.BlockSpec((128, 64), lambda i: (0, 0)),
            pl.BlockSpec((1, 64), lambda i: (0, 0)),
            pl.BlockSpec((64, 16), lambda i: (0, 0)),
            pl.BlockSpec((1, 16), lambda i: (0, 0)),
            pl.BlockSpec((16, 16), lambda i: (0, 0)),
            pl.BlockSpec((1, 16), lambda i: (0, 0)),
            pl.BlockSpec((1, 16), lambda i: (0, 0)),
            pl.BlockSpec((1, 16), lambda i: (0, 0)),
            pl.BlockSpec((128, 128), lambda i: (0, 0)),
            pl.BlockSpec((1, 128), lambda i: (0, 0)),
            pl.BlockSpec((1, 128), lambda i: (0, 0)),
        ],
        out_specs=[
            pl.BlockSpec((R, 128), lambda i: (i, 0)),
            _ST_SPEC,
            _ST_SPEC,
        ],
        out_shape=[
            jax.ShapeDtypeStruct((N_PAD, 128), _f32),
            _ST_SHAPE,
            _ST_SHAPE,
        ],
    )(num4, b4, wm1t, bm1, wm2t, bm2, wm3t, bm3, wm4d, bm4ds, wt5, asv, adv)


def _tc_final(num8, b8):
    def body(n_ref, b_ref, o_ref):
        o_ref[...] = jnp.tanh(n_ref[...] + b_ref[...])

    return pl.pallas_call(
        body,
        grid=(N_PAD // R,),
        in_specs=[
            pl.BlockSpec((R, 128), lambda i: (i, 0)),
            pl.BlockSpec((1, 128), lambda i: (0, 0)),
        ],
        out_specs=pl.BlockSpec((R, 128), lambda i: (i, 0)),
        out_shape=jax.ShapeDtypeStruct((N_PAD, 128), _f32),
    )(num8, b8)


def kernel(x, edge_index, batch, epoch, params):
    src32 = edge_index[0].astype(_i32)
    dst32 = edge_index[1].astype(_i32)
    dsts_s, srcs_s = lax.sort((dst32, src32), num_keys=1)
    pad_d = jnp.full((E_PAD - E_RAW,), N_RAW + 100, _i32)
    pad_s = jnp.zeros((E_PAD - E_RAW,), _i32)
    dsts_p = jnp.concatenate([dsts_s, pad_d])
    srcs_p = jnp.concatenate([srcs_s, pad_s])
    rowptr = jnp.searchsorted(dsts_p, jnp.arange(RP_LEN, dtype=_i32),
                              side="left").astype(_i32)
    cuts = dsts_s[(jnp.arange(1, 32) * E_RAW) // 32]
    nbmid = (cuts // 64) * 64
    nb = jnp.concatenate([
        jnp.zeros((1,), _i32), nbmid.astype(_i32),
        jnp.full((NB_LEN - 32,), N_PAD, _i32)])

    def row(v, w=128):
        out = jnp.zeros((1, w), _f32)
        return out.at[0, : v.shape[0]].set(v)

    wts = {}
    for i in range(1, 9):
        wts[i] = params["Wc%d" % i].T
    w1t = jnp.zeros((48, 128), _f32).at[:42].set(wts[1])
    w8t = jnp.zeros((128, 128), _f32).at[:, :42].set(wts[8])
    asv = {i: row(params["asrc%d" % i]) for i in range(1, 9)}
    adv = {i: row(params["adst%d" % i]) for i in range(1, 9)}
    bs = {i: row(params["bc%d" % i]) for i in range(1, 9)}
    wm1t, wm2t, wm3t = (params["Wm1"].T, params["Wm2"].T, params["Wm3"].T)
    bm1, bm2, bm3 = (row(params["bm1"], 64), row(params["bm2"], 16),
                     row(params["bm3"], 16))
    wm4d = (params["Wm4"][1] - params["Wm4"][0]).reshape(1, 16)
    bm4d = params["bm4"][1] - params["bm4"][0]
    bm4ds = jnp.full((1, 16), bm4d / 16.0, _f32)

    xp = jnp.zeros((N_PAD, 48), _f32).at[:N_RAW, :42].set(x)

    def agg(h, s, t):
        flat = _sc_gat_128(h, s.reshape(-1), t.reshape(-1), srcs_p, dsts_p,
                           rowptr, nb)
        return flat.reshape(N_PAD, 128)

    h, s, t = _tc_first(xp, w1t, asv[1], adv[1])
    num = agg(h, s, t)
    for i in (2, 3, 4):
        h, s, t = _tc_mid(num, bs[i - 1], wts[i], asv[i], adv[i], 128)
        num = agg(h, s, t)
    h, s, t = _tc_mask(num, bs[4], wm1t, bm1, wm2t, bm2, wm3t, bm3,
                       wm4d, bm4ds, wts[5], asv[5], adv[5])
    num = agg(h, s, t)
    for i in (6, 7):
        h, s, t = _tc_mid(num, bs[i - 1], wts[i], asv[i], adv[i], 128)
        num = agg(h, s, t)
    h, s, t = _tc_mid(num, bs[7], w8t, asv[8], adv[8], 128)
    num = agg(h, s, t)
    out = _tc_final(num, bs[8])
    return out[:N_RAW, :42]


# speedup vs baseline: 12.2045x; 1.1152x over previous
import dataclasses
import functools

import jax
import jax.numpy as jnp
from jax import lax
from jax.experimental import pallas as pl
from jax.experimental.pallas import tpu as pltpu
from jax.experimental.pallas import tpu_sc as plsc

N_RAW = 50000
E_RAW = 800000
N_PAD = 50176
CE = 64
E_PAD = E_RAW + 4 * CE
RP_LEN = N_PAD + 128
NB_LEN = 48
NS = 64
R = 512
EPS = 1e-16

_f32 = jnp.float32
_i32 = jnp.int32


def _vsplat(v16, j):
    idx = jnp.full((16,), j, _i32)
    return v16.at[idx].get(mode="promise_in_bounds")


def _make_sc_gat(FC):
    F = FC * 16
    mesh = plsc.VectorSubcoreMesh(core_axis_name="c", subcore_axis_name="s")

    cp = pltpu.CompilerParams()
    if "needs_layout_passes" in pltpu.CompilerParams.__dataclass_fields__:
        cp = dataclasses.replace(cp, needs_layout_passes=False)

    @functools.partial(
        pl.kernel,
        mesh=mesh,
        compiler_params=cp,
        out_type=jax.ShapeDtypeStruct((N_PAD * F,), _f32),
        scratch_types=[
            pltpu.VMEM((N_PAD,), _f32),
            pltpu.VMEM((NS,), _f32),
            pltpu.VMEM((CE,), _i32),
            pltpu.VMEM((CE,), _i32),
            pltpu.VMEM((CE,), _i32),
            pltpu.VMEM((CE,), _i32),
            pltpu.VMEM((CE, F), _f32),
            pltpu.VMEM((CE, F), _f32),
            pltpu.VMEM((NS * F,), _f32),
            pltpu.VMEM((NS,), _f32),
            pltpu.VMEM((96,), _i32),
            pltpu.VMEM((NB_LEN,), _i32),
            pltpu.SemaphoreType.DMA,
            pltpu.SemaphoreType.DMA,
            pltpu.SemaphoreType.DMA,
            pltpu.SemaphoreType.DMA,
            pltpu.SemaphoreType.DMA,
            pltpu.SemaphoreType.DMA,
        ],
    )
    def sc_gat(h_hbm, s_hbm, t_hbm, srcs_hbm, dsts_hbm, rp_hbm, nb_hbm,
               num_hbm,
               s_v, t_v, si0_v, si1_v, dl0_v, dl1_v, hr0_v, hr1_v,
               acc_v, dac_v, rp_v, nb_v,
               semi0, semi1, semd0, semd1, semg0, semg1):
        lane = lax.iota(_i32, 16)
        zero16 = jnp.zeros((16,), _f32)
        wid = lax.axis_index("s") * 2 + lax.axis_index("c")
        si_b = (si0_v, si1_v)
        dl_b = (dl0_v, dl1_v)
        hr_b = (hr0_v, hr1_v)
        semi_b = (semi0, semi1)
        semd_b = (semd0, semd1)
        semg_b = (semg0, semg1)

        pltpu.sync_copy(nb_hbm, nb_v)
        pltpu.sync_copy(s_hbm, s_v)

        nb0 = nb_v[pl.ds(0, 16)]
        nb1 = nb_v[pl.ds(16, 16)]
        nb2 = nb_v[pl.ds(32, 16)]

        def nbsel(i):
            q = i // 16
            r = i - q * 16
            sel = jnp.where(q == 0, nb0, jnp.where(q == 1, nb1, nb2))
            return lax.reduce_max(jnp.where(lane == r, sel, 0), (0,))

        n_lo = nbsel(wid)
        n_hi = nbsel(wid + 1)
        nslices = (n_hi - n_lo) // NS

        def issue_sidl(b, base):
            pltpu.async_copy(srcs_hbm.at[pl.ds(base, CE)], si_b[b], semi_b[b])
            pltpu.async_copy(dsts_hbm.at[pl.ds(base, CE)], dl_b[b], semd_b[b])

        def wait_sidl(b):
            pltpu.make_async_copy(srcs_hbm.at[pl.ds(0, CE)], si_b[b],
                                  semi_b[b]).wait()
            pltpu.make_async_copy(dsts_hbm.at[pl.ds(0, CE)], dl_b[b],
                                  semd_b[b]).wait()

        def issue_g(b):
            pltpu.async_copy(h_hbm.at[si_b[b]], hr_b[b], semg_b[b])

        def wait_g(b):
            pltpu.make_async_copy(h_hbm.at[pl.ds(0, CE)], hr_b[b],
                                  semg_b[b]).wait()

        def slice_body(sl, carry):
            m0 = pl.multiple_of(n_lo + sl * NS, 64)

            @pl.loop(0, NS * F, step=16)
            def _zn(i):
                acc_v[pl.ds(i, 16)] = zero16

            dac_v[pl.ds(0, 16)] = zero16
            dac_v[pl.ds(16, 16)] = zero16
            dac_v[pl.ds(32, 16)] = zero16
            dac_v[pl.ds(48, 16)] = zero16

            pltpu.sync_copy(t_hbm.at[pl.ds(m0, NS)], t_v)
            pltpu.sync_copy(rp_hbm.at[pl.ds(m0, 96)], rp_v)
            rpa = rp_v[pl.ds(0, 16)]
            rpb = rp_v[pl.ds(64, 16)]
            rp_lo = lax.reduce_max(jnp.where(lane == 0, rpa, 0), (0,))
            rp_hi = lax.reduce_max(jnp.where(lane == 0, rpb, 0), (0,))
            c0 = rp_lo // CE
            nch = (rp_hi + (CE - 1)) // CE - c0

            def cbase(ci):
                return pl.multiple_of((c0 + ci) * CE, 64)

            def compute(b, ci):
                ee_l = []
                dc_l = []
                for q in range(CE // 16):
                    si16 = si_b[b][pl.ds(q * 16, 16)]
                    sg = plsc.load_gather(s_v, [si16])
                    d16 = dl_b[b][pl.ds(q * 16, 16)] - m0
                    valid = (d16 >= 0) & (d16 < NS)
                    dc16 = jnp.clip(d16, 0, NS - 1)
                    tg = plsc.load_gather(t_v, [dc16])
                    z = sg + tg
                    zl = jnp.where(z > 0.0, z, 0.2 * z)
                    ee_l.append(jnp.where(valid, jnp.exp(zl), 0.0))
                    dc_l.append(dc16)

                mask0 = lane == 0
                for q in range(CE // 16):
                    for r_ in range(16):
                        j = q * 16 + r_
                        w16 = _vsplat(ee_l[q], r_)
                        d16s = _vsplat(dc_l[q], r_)
                        rowb = d16s * F
                        for k in range(FC):
                            idx = rowb + (lane + (k * 16))
                            vals = w16 * hr_b[b][j, pl.ds(k * 16, 16)]
                            plsc.addupdate_scatter(acc_v, [idx], vals)
                        plsc.addupdate_scatter(dac_v, [d16s], w16, mask=mask0)

            @pl.when(nch > 0)
            def _pro():
                issue_sidl(0, cbase(0))

                @pl.when(nch > 1)
                def _pro1():
                    issue_sidl(1, cbase(1))

                wait_sidl(0)
                issue_g(0)

            def step(b, ci):
                @pl.when(ci < nch)
                def _s():
                    wait_g(b)

                    @pl.when(ci + 1 < nch)
                    def _nx():
                        wait_sidl(1 - b)
                        issue_g(1 - b)

                    compute(b, ci)

                    @pl.when(ci + 2 < nch)
                    def _pf():
                        issue_sidl(b, cbase(ci + 2))

            def chunk_pair(i2, ccarry):
                step(0, 2 * i2)
                step(1, 2 * i2 + 1)
                return ccarry

            lax.fori_loop(0, (nch + 1) // 2, chunk_pair, 0)

            for g in range(NS // 16):
                den16 = dac_v[pl.ds(g * 16, 16)]
                rec16 = 1.0 / (den16 + EPS)
                for r_ in range(16):
                    rr = g * 16 + r_
                    rec = _vsplat(rec16, r_)
                    for k in range(FC):
                        off = rr * F + k * 16
                        acc_v[pl.ds(off, 16)] = rec * acc_v[pl.ds(off, 16)]

            pltpu.sync_copy(acc_v, num_hbm.at[pl.ds(m0 * F, NS * F)])
            return carry

        lax.fori_loop(0, nslices, slice_body, 0)

    return sc_gat


_sc_gat_128 = _make_sc_gat(8)


def _st_out(h, asv, adv, s_ref, t_ref):
    s_ref[...] = jnp.sum(h * asv, axis=1).reshape(1, 4, 128)
    t_ref[...] = jnp.sum(h * adv, axis=1).reshape(1, 4, 128)


_ST_SPEC = pl.BlockSpec((1, 4, 128), lambda i: (i, 0, 0))
_ST_SHAPE = jax.ShapeDtypeStruct((N_PAD // R, 4, 128), _f32)


def _tc_first(x, wt, asv, adv):
    din = x.shape[1]

    def body(x_ref, w_ref, as_ref, ad_ref, h_ref, s_ref, t_ref):
        xb = x_ref[...]
        xin = xb / jnp.maximum(jnp.sum(jnp.abs(xb), axis=1, keepdims=True),
                               1e-12)
        h = jnp.dot(xin, w_ref[...], preferred_element_type=_f32)
        h_ref[...] = h
        _st_out(h, as_ref[...], ad_ref[...], s_ref, t_ref)

    return pl.pallas_call(
        body,
        grid=(N_PAD // R,),
        in_specs=[
            pl.BlockSpec((R, din), lambda i: (i, 0)),
            pl.BlockSpec((din, 128), lambda i: (0, 0)),
            pl.BlockSpec((1, 128), lambda i: (0, 0)),
            pl.BlockSpec((1, 128), lambda i: (0, 0)),
        ],
        out_specs=[
            pl.BlockSpec((R, 128), lambda i: (i, 0)),
            _ST_SPEC,
            _ST_SPEC,
        ],
        out_shape=[
            jax.ShapeDtypeStruct((N_PAD, 128), _f32),
            _ST_SHAPE,
            _ST_SHAPE,
        ],
    )(x, wt, asv, adv)


def _tc_mid(num, bprev, wt, asv, adv, dout):
    def body(n_ref, b_ref, w_ref, as_ref, ad_ref, h_ref, s_ref, t_ref):
        xin = jnp.tanh(n_ref[...] + b_ref[...])
        h = jnp.dot(xin, w_ref[...], preferred_element_type=_f32)
        h_ref[...] = h
        _st_out(h, as_ref[...], ad_ref[...], s_ref, t_ref)

    return pl.pallas_call(
        body,
        grid=(N_PAD // R,),
        in_specs=[
            pl.BlockSpec((R, 128), lambda i: (i, 0)),
            pl.BlockSpec((1, 128), lambda i: (0, 0)),
            pl.BlockSpec((128, dout), lambda i: (0, 0)),
            pl.BlockSpec((1, dout), lambda i: (0, 0)),
            pl.BlockSpec((1, dout), lambda i: (0, 0)),
        ],
        out_specs=[
            pl.BlockSpec((R, dout), lambda i: (i, 0)),
            _ST_SPEC,
            _ST_SPEC,
        ],
        out_shape=[
            jax.ShapeDtypeStruct((N_PAD, dout), _f32),
            _ST_SHAPE,
            _ST_SHAPE,
        ],
    )(num, bprev, wt, asv, adv)


def _tc_mask(num4, b4, wm1t, bm1, wm2t, bm2, wm3t, bm3, wm4d, bm4ds,
             wt5, asv, adv):
    def body(n_ref, b_ref, w1_ref, b1_ref, w2_ref, b2_ref, w3_ref, b3_ref,
             w4_ref, b4d_ref, w5_ref, as_ref, ad_ref, h_ref, s_ref, t_ref):
        latent = jnp.tanh(n_ref[...] + b_ref[...])
        m = jnp.tanh(jnp.dot(latent, w1_ref[...],
                             preferred_element_type=_f32) + b1_ref[...])
        m = jnp.tanh(jnp.dot(m, w2_ref[...],
                             preferred_element_type=_f32) + b2_ref[...])
        m = jnp.tanh(jnp.dot(m, w3_ref[...],
                             preferred_element_type=_f32) + b3_ref[...])
        logit = jnp.sum(m * w4_ref[...] + b4d_ref[...], axis=1,
                        keepdims=True)
        sa = jax.nn.sigmoid(logit)
        d0 = latent * sa
        h = jnp.dot(d0, w5_ref[...], preferred_element_type=_f32)
        h_ref[...] = h
        _st_out(h, as_ref[...], ad_ref[...], s_ref, t_ref)

    return pl.pallas_call(
        body,
        grid=(N_PAD // R,),
        in_specs=[
            pl.BlockSpec((R, 128), lambda i: (i, 0)),
            pl.BlockSpec((1, 128), lambda i: (0, 0)),
            pl.BlockSpec((128, 64), lambda i: (0, 0)),
            pl.BlockSpec((1, 64), lambda i: (0, 0)),
            pl.BlockSpec((64, 16), lambda i: (0, 0)),
            pl.BlockSpec((1, 16), lambda i: (0, 0)),
            pl.BlockSpec((16, 16), lambda i: (0, 0)),
            pl.BlockSpec((1, 16), lambda i: (0, 0)),
            pl.BlockSpec((1, 16), lambda i: (0, 0)),
            pl.BlockSpec((1, 16), lambda i: (0, 0)),
            pl.BlockSpec((128, 128), lambda i: (0, 0)),
            pl.BlockSpec((1, 128), lambda i: (0, 0)),
            pl.BlockSpec((1, 128), lambda i: (0, 0)),
        ],
        out_specs=[
            pl.BlockSpec((R, 128), lambda i: (i, 0)),
            _ST_SPEC,
            _ST_SPEC,
        ],
        out_shape=[
            jax.ShapeDtypeStruct((N_PAD, 128), _f32),
            _ST_SHAPE,
            _ST_SHAPE,
        ],
    )(num4, b4, wm1t, bm1, wm2t, bm2, wm3t, bm3, wm4d, bm4ds, wt5, asv, adv)


def _tc_final(num8, b8):
    def body(n_ref, b_ref, o_ref):
        o_ref[...] = jnp.tanh(n_ref[...] + b_ref[...])

    return pl.pallas_call(
        body,
        grid=(N_PAD // R,),
        in_specs=[
            pl.BlockSpec((R, 128), lambda i: (i, 0)),
            pl.BlockSpec((1, 128), lambda i: (0, 0)),
        ],
        out_specs=pl.BlockSpec((R, 128), lambda i: (i, 0)),
        out_shape=jax.ShapeDtypeStruct((N_PAD, 128), _f32),
    )(num8, b8)


def kernel(x, edge_index, batch, epoch, params):
    src32 = edge_index[0].astype(_i32)
    dst32 = edge_index[1].astype(_i32)
    dsts_s, srcs_s = lax.sort((dst32, src32), num_keys=1)
    pad_d = jnp.full((E_PAD - E_RAW,), N_RAW + 100, _i32)
    pad_s = jnp.zeros((E_PAD - E_RAW,), _i32)
    dsts_p = jnp.concatenate([dsts_s, pad_d])
    srcs_p = jnp.concatenate([srcs_s, pad_s])
    rowptr = jnp.searchsorted(dsts_p, jnp.arange(RP_LEN, dtype=_i32),
                              side="left").astype(_i32)
    cuts = dsts_s[(jnp.arange(1, 32) * E_RAW) // 32]
    nbmid = (cuts // 64) * 64
    nb = jnp.concatenate([
        jnp.zeros((1,), _i32), nbmid.astype(_i32),
        jnp.full((NB_LEN - 32,), N_PAD, _i32)])

    def row(v, w=128):
        out = jnp.zeros((1, w), _f32)
        return out.at[0, : v.shape[0]].set(v)

    wts = {}
    for i in range(1, 9):
        wts[i] = params["Wc%d" % i].T
    w1t = jnp.zeros((48, 128), _f32).at[:42].set(wts[1])
    w8t = jnp.zeros((128, 128), _f32).at[:, :42].set(wts[8])
    asv = {i: row(params["asrc%d" % i]) for i in range(1, 9)}
    adv = {i: row(params["adst%d" % i]) for i in range(1, 9)}
    bs = {i: row(params["bc%d" % i]) for i in range(1, 9)}
    wm1t, wm2t, wm3t = (params["Wm1"].T, params["Wm2"].T, params["Wm3"].T)
    bm1, bm2, bm3 = (row(params["bm1"], 64), row(params["bm2"], 16),
                     row(params["bm3"], 16))
    wm4d = (params["Wm4"][1] - params["Wm4"][0]).reshape(1, 16)
    bm4d = params["bm4"][1] - params["bm4"][0]
    bm4ds = jnp.full((1, 16), bm4d / 16.0, _f32)

    xp = jnp.zeros((N_PAD, 48), _f32).at[:N_RAW, :42].set(x)

    def agg(h, s, t):
        flat = _sc_gat_128(h, s.reshape(-1), t.reshape(-1), srcs_p, dsts_p,
                           rowptr, nb)
        return flat.reshape(N_PAD, 128)

    h, s, t = _tc_first(xp, w1t, asv[1], adv[1])
    num = agg(h, s, t)
    for i in (2, 3, 4):
        h, s, t = _tc_mid(num, bs[i - 1], wts[i], asv[i], adv[i], 128)
        num = agg(h, s, t)
    h, s, t = _tc_mask(num, bs[4], wm1t, bm1, wm2t, bm2, wm3t, bm3,
                       wm4d, bm4ds, wts[5], asv[5], adv[5])
    num = agg(h, s, t)
    for i in (6, 7):
        h, s, t = _tc_mid(num, bs[i - 1], wts[i], asv[i], adv[i], 128)
        num = agg(h, s, t)
    h, s, t = _tc_mid(num, bs[7], w8t, asv[8], adv[8], 128)
    num = agg(h, s, t)
    out = _tc_final(num, bs[8])
    return out[:N_RAW, :42]


# speedup vs baseline: 12.4385x; 1.0192x over previous
import dataclasses
import functools

import jax
import jax.numpy as jnp
from jax import lax
from jax.experimental import pallas as pl
from jax.experimental.pallas import tpu as pltpu
from jax.experimental.pallas import tpu_sc as plsc

N_RAW = 50000
E_RAW = 800000
N_PAD = 50176
CE = 64
E_PAD = E_RAW + 4 * CE
RP_LEN = N_PAD + 128
NB_LEN = 48
NS = 64
R = 512
EPS = 1e-16

_f32 = jnp.float32
_i32 = jnp.int32


def _vsplat(v16, j):
    idx = jnp.full((16,), j, _i32)
    return v16.at[idx].get(mode="promise_in_bounds")


def _make_sc_gat(FC):
    F = FC * 16
    mesh = plsc.VectorSubcoreMesh(core_axis_name="c", subcore_axis_name="s")

    cp = pltpu.CompilerParams()
    if "needs_layout_passes" in pltpu.CompilerParams.__dataclass_fields__:
        cp = dataclasses.replace(cp, needs_layout_passes=False)

    @functools.partial(
        pl.kernel,
        mesh=mesh,
        compiler_params=cp,
        out_type=jax.ShapeDtypeStruct((N_PAD, F), _f32),
        scratch_types=[
            pltpu.VMEM((N_PAD,), _f32),
            pltpu.VMEM((NS,), _f32),
            pltpu.VMEM((CE,), _i32),
            pltpu.VMEM((CE,), _i32),
            pltpu.VMEM((CE,), _i32),
            pltpu.VMEM((CE,), _i32),
            pltpu.VMEM((CE, F), _f32),
            pltpu.VMEM((CE, F), _f32),
            pltpu.VMEM((NS, F), _f32),
            pltpu.VMEM((NS,), _f32),
            pltpu.VMEM((96,), _i32),
            pltpu.VMEM((NB_LEN,), _i32),
            pltpu.SemaphoreType.DMA,
            pltpu.SemaphoreType.DMA,
            pltpu.SemaphoreType.DMA,
            pltpu.SemaphoreType.DMA,
            pltpu.SemaphoreType.DMA,
            pltpu.SemaphoreType.DMA,
        ],
    )
    def sc_gat(h_hbm, s_hbm, t_hbm, srcs_hbm, dsts_hbm, rp_hbm, nb_hbm,
               zer_hbm,
               num_hbm,
               s_v, t_v, si0_v, si1_v, dl0_v, dl1_v, hr0_v, hr1_v,
               acc_v, dac_v, rp_v, nb_v,
               semi0, semi1, semd0, semd1, semg0, semg1):
        lane = lax.iota(_i32, 16)
        zero16 = jnp.zeros((16,), _f32)
        wid = lax.axis_index("s") * 2 + lax.axis_index("c")
        si_b = (si0_v, si1_v)
        dl_b = (dl0_v, dl1_v)
        hr_b = (hr0_v, hr1_v)
        semi_b = (semi0, semi1)
        semd_b = (semd0, semd1)
        semg_b = (semg0, semg1)

        pltpu.sync_copy(nb_hbm, nb_v)
        pltpu.sync_copy(s_hbm, s_v)

        nb0 = nb_v[pl.ds(0, 16)]
        nb1 = nb_v[pl.ds(16, 16)]
        nb2 = nb_v[pl.ds(32, 16)]

        def nbsel(i):
            q = i // 16
            r = i - q * 16
            sel = jnp.where(q == 0, nb0, jnp.where(q == 1, nb1, nb2))
            return lax.reduce_max(jnp.where(lane == r, sel, 0), (0,))

        n_lo = nbsel(wid)
        n_hi = nbsel(wid + 1)
        nslices = (n_hi - n_lo) // NS

        def issue_sidl(b, base):
            pltpu.async_copy(srcs_hbm.at[pl.ds(base, CE)], si_b[b], semi_b[b])
            pltpu.async_copy(dsts_hbm.at[pl.ds(base, CE)], dl_b[b], semd_b[b])

        def wait_sidl(b):
            pltpu.make_async_copy(srcs_hbm.at[pl.ds(0, CE)], si_b[b],
                                  semi_b[b]).wait()
            pltpu.make_async_copy(dsts_hbm.at[pl.ds(0, CE)], dl_b[b],
                                  semd_b[b]).wait()

        def issue_g(b):
            pltpu.async_copy(h_hbm.at[si_b[b]], hr_b[b], semg_b[b])

        def wait_g(b):
            pltpu.make_async_copy(h_hbm.at[pl.ds(0, CE)], hr_b[b],
                                  semg_b[b]).wait()

        def slice_body(sl, carry):
            m0 = pl.multiple_of(n_lo + sl * NS, 64)

            pltpu.sync_copy(zer_hbm, acc_v)

            dac_v[pl.ds(0, 16)] = zero16
            dac_v[pl.ds(16, 16)] = zero16
            dac_v[pl.ds(32, 16)] = zero16
            dac_v[pl.ds(48, 16)] = zero16

            pltpu.sync_copy(t_hbm.at[pl.ds(m0, NS)], t_v)
            pltpu.sync_copy(rp_hbm.at[pl.ds(m0, 96)], rp_v)
            rpa = rp_v[pl.ds(0, 16)]
            rpb = rp_v[pl.ds(64, 16)]
            rp_lo = lax.reduce_max(jnp.where(lane == 0, rpa, 0), (0,))
            rp_hi = lax.reduce_max(jnp.where(lane == 0, rpb, 0), (0,))
            c0 = rp_lo // CE
            nch = (rp_hi + (CE - 1)) // CE - c0

            def cbase(ci):
                return pl.multiple_of((c0 + ci) * CE, 64)

            cols = [lane + (k * 16) for k in range(FC)]
            mask0 = lane == 0

            def compute_ee(b):
                ee_l = []
                dc_l = []
                for q in range(CE // 16):
                    si16 = si_b[b][pl.ds(q * 16, 16)]
                    sg = plsc.load_gather(s_v, [si16])
                    d16 = dl_b[b][pl.ds(q * 16, 16)] - m0
                    valid = (d16 >= 0) & (d16 < NS)
                    dc16 = jnp.clip(d16, 0, NS - 1)
                    tg = plsc.load_gather(t_v, [dc16])
                    z = sg + tg
                    zl = jnp.where(z > 0.0, z, 0.2 * z)
                    ee_l.append(jnp.where(valid, jnp.exp(zl), 0.0))
                    dc_l.append(dc16)
                return ee_l, dc_l

            def accumulate(b, ee_l, dc_l):
                for q in range(CE // 16):
                    for r_ in range(16):
                        j = q * 16 + r_
                        w16 = _vsplat(ee_l[q], r_)
                        d16s = _vsplat(dc_l[q], r_)
                        for k in range(FC):
                            vals = w16 * hr_b[b][j, pl.ds(k * 16, 16)]
                            plsc.addupdate_scatter(acc_v, [d16s, cols[k]],
                                                   vals)
                        plsc.addupdate_scatter(dac_v, [d16s], w16, mask=mask0)

            @pl.when(nch > 0)
            def _pro():
                issue_sidl(0, cbase(0))

                @pl.when(nch > 1)
                def _pro1():
                    issue_sidl(1, cbase(1))

                wait_sidl(0)
                issue_g(0)

            def step(b, ci):
                @pl.when(ci < nch)
                def _s():
                    ee_l, dc_l = compute_ee(b)
                    wait_g(b)

                    @pl.when(ci + 1 < nch)
                    def _nx():
                        wait_sidl(1 - b)
                        issue_g(1 - b)

                    accumulate(b, ee_l, dc_l)

                    @pl.when(ci + 2 < nch)
                    def _pf():
                        issue_sidl(b, cbase(ci + 2))

            def chunk_pair(i2, ccarry):
                step(0, 2 * i2)
                step(1, 2 * i2 + 1)
                return ccarry

            lax.fori_loop(0, (nch + 1) // 2, chunk_pair, 0)

            for g in range(NS // 16):
                den16 = dac_v[pl.ds(g * 16, 16)]
                rec16 = 1.0 / (den16 + EPS)
                for r_ in range(16):
                    rr = g * 16 + r_
                    rec = _vsplat(rec16, r_)
                    for k in range(FC):
                        acc_v[rr, pl.ds(k * 16, 16)] = (
                            rec * acc_v[rr, pl.ds(k * 16, 16)])

            pltpu.sync_copy(acc_v, num_hbm.at[pl.ds(m0, NS)])
            return carry

        lax.fori_loop(0, nslices, slice_body, 0)

    return sc_gat


_sc_gat_128 = _make_sc_gat(8)


def _st_out(h, asv, adv, s_ref, t_ref):
    s_ref[...] = jnp.sum(h * asv, axis=1).reshape(1, 4, 128)
    t_ref[...] = jnp.sum(h * adv, axis=1).reshape(1, 4, 128)


_ST_SPEC = pl.BlockSpec((1, 4, 128), lambda i: (i, 0, 0))
_ST_SHAPE = jax.ShapeDtypeStruct((N_PAD // R, 4, 128), _f32)


def _tc_first(x, wt, asv, adv):
    din = x.shape[1]

    def body(x_ref, w_ref, as_ref, ad_ref, h_ref, s_ref, t_ref):
        xb = x_ref[...]
        xin = xb / jnp.maximum(jnp.sum(jnp.abs(xb), axis=1, keepdims=True),
                               1e-12)
        h = jnp.dot(xin, w_ref[...], preferred_element_type=_f32)
        h_ref[...] = h
        _st_out(h, as_ref[...], ad_ref[...], s_ref, t_ref)

    return pl.pallas_call(
        body,
        grid=(N_PAD // R,),
        in_specs=[
            pl.BlockSpec((R, din), lambda i: (i, 0)),
            pl.BlockSpec((din, 128), lambda i: (0, 0)),
            pl.BlockSpec((1, 128), lambda i: (0, 0)),
            pl.BlockSpec((1, 128), lambda i: (0, 0)),
        ],
        out_specs=[
            pl.BlockSpec((R, 128), lambda i: (i, 0)),
            _ST_SPEC,
            _ST_SPEC,
        ],
        out_shape=[
            jax.ShapeDtypeStruct((N_PAD, 128), _f32),
            _ST_SHAPE,
            _ST_SHAPE,
        ],
    )(x, wt, asv, adv)


def _tc_mid(num, bprev, wt, asv, adv, dout):
    def body(n_ref, b_ref, w_ref, as_ref, ad_ref, h_ref, s_ref, t_ref):
        xin = jnp.tanh(n_ref[...] + b_ref[...])
        h = jnp.dot(xin, w_ref[...], preferred_element_type=_f32)
        h_ref[...] = h
        _st_out(h, as_ref[...], ad_ref[...], s_ref, t_ref)

    return pl.pallas_call(
        body,
        grid=(N_PAD // R,),
        in_specs=[
            pl.BlockSpec((R, 128), lambda i: (i, 0)),
            pl.BlockSpec((1, 128), lambda i: (0, 0)),
            pl.BlockSpec((128, dout), lambda i: (0, 0)),
            pl.BlockSpec((1, dout), lambda i: (0, 0)),
            pl.BlockSpec((1, dout), lambda i: (0, 0)),
        ],
        out_specs=[
            pl.BlockSpec((R, dout), lambda i: (i, 0)),
            _ST_SPEC,
            _ST_SPEC,
        ],
        out_shape=[
            jax.ShapeDtypeStruct((N_PAD, dout), _f32),
            _ST_SHAPE,
            _ST_SHAPE,
        ],
    )(num, bprev, wt, asv, adv)


def _tc_mask(num4, b4, wm1t, bm1, wm2t, bm2, wm3t, bm3, wm4d, bm4ds,
             wt5, asv, adv):
    def body(n_ref, b_ref, w1_ref, b1_ref, w2_ref, b2_ref, w3_ref, b3_ref,
             w4_ref, b4d_ref, w5_ref, as_ref, ad_ref, h_ref, s_ref, t_ref):
        latent = jnp.tanh(n_ref[...] + b_ref[...])
        m = jnp.tanh(jnp.dot(latent, w1_ref[...],
                             preferred_element_type=_f32) + b1_ref[...])
        m = jnp.tanh(jnp.dot(m, w2_ref[...],
                             preferred_element_type=_f32) + b2_ref[...])
        m = jnp.tanh(jnp.dot(m, w3_ref[...],
                             preferred_element_type=_f32) + b3_ref[...])
        logit = jnp.sum(m * w4_ref[...] + b4d_ref[...], axis=1,
                        keepdims=True)
        sa = jax.nn.sigmoid(logit)
        d0 = latent * sa
        h = jnp.dot(d0, w5_ref[...], preferred_element_type=_f32)
        h_ref[...] = h
        _st_out(h, as_ref[...], ad_ref[...], s_ref, t_ref)

    return pl.pallas_call(
        body,
        grid=(N_PAD // R,),
        in_specs=[
            pl.BlockSpec((R, 128), lambda i: (i, 0)),
            pl.BlockSpec((1, 128), lambda i: (0, 0)),
            pl.BlockSpec((128, 64), lambda i: (0, 0)),
            pl.BlockSpec((1, 64), lambda i: (0, 0)),
            pl.BlockSpec((64, 16), lambda i: (0, 0)),
            pl.BlockSpec((1, 16), lambda i: (0, 0)),
            pl.BlockSpec((16, 16), lambda i: (0, 0)),
            pl.BlockSpec((1, 16), lambda i: (0, 0)),
            pl.BlockSpec((1, 16), lambda i: (0, 0)),
            pl.BlockSpec((1, 16), lambda i: (0, 0)),
            pl.BlockSpec((128, 128), lambda i: (0, 0)),
            pl.BlockSpec((1, 128), lambda i: (0, 0)),
            pl.BlockSpec((1, 128), lambda i: (0, 0)),
        ],
        out_specs=[
            pl.BlockSpec((R, 128), lambda i: (i, 0)),
            _ST_SPEC,
            _ST_SPEC,
        ],
        out_shape=[
            jax.ShapeDtypeStruct((N_PAD, 128), _f32),
            _ST_SHAPE,
            _ST_SHAPE,
        ],
    )(num4, b4, wm1t, bm1, wm2t, bm2, wm3t, bm3, wm4d, bm4ds, wt5, asv, adv)


def _tc_final(num8, b8):
    def body(n_ref, b_ref, o_ref):
        o_ref[...] = jnp.tanh(n_ref[...] + b_ref[...])

    return pl.pallas_call(
        body,
        grid=(N_PAD // R,),
        in_specs=[
            pl.BlockSpec((R, 128), lambda i: (i, 0)),
            pl.BlockSpec((1, 128), lambda i: (0, 0)),
        ],
        out_specs=pl.BlockSpec((R, 128), lambda i: (i, 0)),
        out_shape=jax.ShapeDtypeStruct((N_PAD, 128), _f32),
    )(num8, b8)


def kernel(x, edge_index, batch, epoch, params):
    src32 = edge_index[0].astype(_i32)
    dst32 = edge_index[1].astype(_i32)
    dsts_s, srcs_s = lax.sort((dst32, src32), num_keys=1)
    pad_d = jnp.full((E_PAD - E_RAW,), N_RAW + 100, _i32)
    pad_s = jnp.zeros((E_PAD - E_RAW,), _i32)
    dsts_p = jnp.concatenate([dsts_s, pad_d])
    srcs_p = jnp.concatenate([srcs_s, pad_s])
    rowptr = jnp.searchsorted(dsts_p, jnp.arange(RP_LEN, dtype=_i32),
                              side="left").astype(_i32)
    cuts = dsts_s[(jnp.arange(1, 32) * E_RAW) // 32]
    nbmid = (cuts // 64) * 64
    nb = jnp.concatenate([
        jnp.zeros((1,), _i32), nbmid.astype(_i32),
        jnp.full((NB_LEN - 32,), N_PAD, _i32)])

    def row(v, w=128):
        out = jnp.zeros((1, w), _f32)
        return out.at[0, : v.shape[0]].set(v)

    wts = {}
    for i in range(1, 9):
        wts[i] = params["Wc%d" % i].T
    w1t = jnp.zeros((48, 128), _f32).at[:42].set(wts[1])
    w8t = jnp.zeros((128, 128), _f32).at[:, :42].set(wts[8])
    asv = {i: row(params["asrc%d" % i]) for i in range(1, 9)}
    adv = {i: row(params["adst%d" % i]) for i in range(1, 9)}
    bs = {i: row(params["bc%d" % i]) for i in range(1, 9)}
    wm1t, wm2t, wm3t = (params["Wm1"].T, params["Wm2"].T, params["Wm3"].T)
    bm1, bm2, bm3 = (row(params["bm1"], 64), row(params["bm2"], 16),
                     row(params["bm3"], 16))
    wm4d = (params["Wm4"][1] - params["Wm4"][0]).reshape(1, 16)
    bm4d = params["bm4"][1] - params["bm4"][0]
    bm4ds = jnp.full((1, 16), bm4d / 16.0, _f32)

    xp = jnp.zeros((N_PAD, 48), _f32).at[:N_RAW, :42].set(x)

    zer = jnp.zeros((NS, 128), _f32)

    def agg(h, s, t):
        return _sc_gat_128(h, s.reshape(-1), t.reshape(-1), srcs_p, dsts_p,
                           rowptr, nb, zer)

    h, s, t = _tc_first(xp, w1t, asv[1], adv[1])
    num = agg(h, s, t)
    for i in (2, 3, 4):
        h, s, t = _tc_mid(num, bs[i - 1], wts[i], asv[i], adv[i], 128)
        num = agg(h, s, t)
    h, s, t = _tc_mask(num, bs[4], wm1t, bm1, wm2t, bm2, wm3t, bm3,
                       wm4d, bm4ds, wts[5], asv[5], adv[5])
    num = agg(h, s, t)
    for i in (6, 7):
        h, s, t = _tc_mid(num, bs[i - 1], wts[i], asv[i], adv[i], 128)
        num = agg(h, s, t)
    h, s, t = _tc_mid(num, bs[7], w8t, asv[8], adv[8], 128)
    num = agg(h, s, t)
    out = _tc_final(num, bs[8])
    return out[:N_RAW, :42]


# speedup vs baseline: 22.8127x; 1.8340x over previous
import dataclasses
import functools

import jax
import jax.numpy as jnp
from jax import lax
from jax.experimental import pallas as pl
from jax.experimental.pallas import tpu as pltpu
from jax.experimental.pallas import tpu_sc as plsc

N_RAW = 50000
E_RAW = 800000
N_PAD = 50176
CE = 64
E_PAD = E_RAW + 4 * CE
RP_LEN = N_PAD + 128
NB_LEN = 48
NS = 64
R = 512
EPS = 1e-16

_f32 = jnp.float32
_i32 = jnp.int32


def _vsplat(v16, j):
    idx = jnp.full((16,), j, _i32)
    return v16.at[idx].get(mode="promise_in_bounds")


def _make_sc_gat(FC):
    F = FC * 16
    mesh = plsc.VectorSubcoreMesh(core_axis_name="c", subcore_axis_name="s")

    cp = pltpu.CompilerParams()
    if "needs_layout_passes" in pltpu.CompilerParams.__dataclass_fields__:
        cp = dataclasses.replace(cp, needs_layout_passes=False)

    @functools.partial(
        pl.kernel,
        mesh=mesh,
        compiler_params=cp,
        out_type=jax.ShapeDtypeStruct((N_PAD, F), _f32),
        scratch_types=[
            pltpu.VMEM((N_PAD,), _f32),
            pltpu.VMEM((NS,), _f32),
            pltpu.VMEM((CE,), _i32),
            pltpu.VMEM((CE,), _i32),
            pltpu.VMEM((CE,), _i32),
            pltpu.VMEM((CE,), _i32),
            pltpu.VMEM((CE, F), _f32),
            pltpu.VMEM((CE, F), _f32),
            pltpu.VMEM((CE, F), _f32),
            pltpu.VMEM((CE, F), _f32),
            pltpu.VMEM((CE,), _i32),
            pltpu.VMEM((CE,), _i32),
            pltpu.VMEM((NS, F), _f32),
            pltpu.VMEM_SHARED((16 * NS, F), _f32),
            pltpu.VMEM((NS,), _f32),
            pltpu.VMEM((96,), _i32),
            pltpu.VMEM((NB_LEN,), _i32),
            pltpu.SemaphoreType.DMA,
            pltpu.SemaphoreType.DMA,
            pltpu.SemaphoreType.DMA,
            pltpu.SemaphoreType.DMA,
            pltpu.SemaphoreType.DMA,
            pltpu.SemaphoreType.DMA,
            pltpu.SemaphoreType.DMA,
            pltpu.SemaphoreType.DMA,
        ],
    )
    def sc_gat(h_hbm, s_hbm, t_hbm, srcs_hbm, dsts_hbm, rp_hbm, nb_hbm,
               zer_hbm,
               num_hbm,
               s_v, t_v, si0_v, si1_v, dl0_v, dl1_v, hr0_v, hr1_v,
               sr0_v, sr1_v, dc0_v, dc1_v,
               acc_v, sacc_v, dac_v, rp_v, nb_v,
               semi0, semi1, semd0, semd1, semg0, semg1, semc0, semc1):
        lane = lax.iota(_i32, 16)
        zero16 = jnp.zeros((16,), _f32)
        sid = lax.axis_index("s")
        wid = sid * 2 + lax.axis_index("c")
        srow0 = pl.multiple_of(sid * NS, 64)
        si_b = (si0_v, si1_v)
        dl_b = (dl0_v, dl1_v)
        hr_b = (hr0_v, hr1_v)
        semi_b = (semi0, semi1)
        semd_b = (semd0, semd1)
        semg_b = (semg0, semg1)
        sr_b = (sr0_v, sr1_v)
        dc_b = (dc0_v, dc1_v)
        semc_b = (semc0, semc1)

        pltpu.sync_copy(nb_hbm, nb_v)
        pltpu.sync_copy(s_hbm, s_v)

        nb0 = nb_v[pl.ds(0, 16)]
        nb1 = nb_v[pl.ds(16, 16)]
        nb2 = nb_v[pl.ds(32, 16)]

        def nbsel(i):
            q = i // 16
            r = i - q * 16
            sel = jnp.where(q == 0, nb0, jnp.where(q == 1, nb1, nb2))
            return lax.reduce_max(jnp.where(lane == r, sel, 0), (0,))

        n_lo = nbsel(wid)
        n_hi = nbsel(wid + 1)
        nslices = (n_hi - n_lo) // NS

        def issue_sidl(b, base):
            pltpu.async_copy(srcs_hbm.at[pl.ds(base, CE)], si_b[b], semi_b[b])
            pltpu.async_copy(dsts_hbm.at[pl.ds(base, CE)], dl_b[b], semd_b[b])

        def wait_sidl(b):
            pltpu.make_async_copy(srcs_hbm.at[pl.ds(0, CE)], si_b[b],
                                  semi_b[b]).wait()
            pltpu.make_async_copy(dsts_hbm.at[pl.ds(0, CE)], dl_b[b],
                                  semd_b[b]).wait()

        def issue_g(b):
            pltpu.async_copy(h_hbm.at[si_b[b]], hr_b[b], semg_b[b])

        def wait_g(b):
            pltpu.make_async_copy(h_hbm.at[pl.ds(0, CE)], hr_b[b],
                                  semg_b[b]).wait()

        def issue_sc(b):
            pltpu.async_copy(sr_b[b], sacc_v.at[dc_b[b]], semc_b[b],
                             add=True)

        def wait_sc(b):
            pltpu.make_async_copy(h_hbm.at[pl.ds(0, CE)], sr_b[b],
                                  semc_b[b]).wait()

        def slice_body(sl, carry):
            m0 = pl.multiple_of(n_lo + sl * NS, 64)

            pltpu.sync_copy(zer_hbm, sacc_v.at[pl.ds(srow0, NS)])

            dac_v[pl.ds(0, 16)] = zero16
            dac_v[pl.ds(16, 16)] = zero16
            dac_v[pl.ds(32, 16)] = zero16
            dac_v[pl.ds(48, 16)] = zero16

            pltpu.sync_copy(t_hbm.at[pl.ds(m0, NS)], t_v)
            pltpu.sync_copy(rp_hbm.at[pl.ds(m0, 96)], rp_v)
            rpa = rp_v[pl.ds(0, 16)]
            rpb = rp_v[pl.ds(64, 16)]
            rp_lo = lax.reduce_max(jnp.where(lane == 0, rpa, 0), (0,))
            rp_hi = lax.reduce_max(jnp.where(lane == 0, rpb, 0), (0,))
            c0 = rp_lo // CE
            nch = (rp_hi + (CE - 1)) // CE - c0

            def cbase(ci):
                return pl.multiple_of((c0 + ci) * CE, 64)

            cols = [lane + (k * 16) for k in range(FC)]
            mask0 = lane == 0

            def compute_ee(b):
                ee_l = []
                dc_l = []
                for q in range(CE // 16):
                    si16 = si_b[b][pl.ds(q * 16, 16)]
                    sg = plsc.load_gather(s_v, [si16])
                    d16 = dl_b[b][pl.ds(q * 16, 16)] - m0
                    valid = (d16 >= 0) & (d16 < NS)
                    dc16 = jnp.clip(d16, 0, NS - 1)
                    tg = plsc.load_gather(t_v, [dc16])
                    z = sg + tg
                    zl = jnp.where(z > 0.0, z, 0.2 * z)
                    ee_l.append(jnp.where(valid, jnp.exp(zl), 0.0))
                    dc_l.append(dc16)
                return ee_l, dc_l

            def accumulate(b, ee_l, dc_l):
                for q in range(CE // 16):
                    dc_b[b][pl.ds(q * 16, 16)] = dc_l[q] + srow0
                    for r_ in range(16):
                        j = q * 16 + r_
                        w16 = _vsplat(ee_l[q], r_)
                        d16s = _vsplat(dc_l[q], r_)
                        for k in range(FC):
                            sr_b[b][j, pl.ds(k * 16, 16)] = (
                                w16 * hr_b[b][j, pl.ds(k * 16, 16)])
                        plsc.addupdate_scatter(dac_v, [d16s], w16, mask=mask0)
                issue_sc(b)

            @pl.when(nch > 0)
            def _pro():
                issue_sidl(0, cbase(0))

                @pl.when(nch > 1)
                def _pro1():
                    issue_sidl(1, cbase(1))

                wait_sidl(0)
                issue_g(0)

            def step(b, ci):
                @pl.when(ci < nch)
                def _s():
                    ee_l, dc_l = compute_ee(b)
                    wait_g(b)

                    @pl.when(ci + 1 < nch)
                    def _nx():
                        wait_sidl(1 - b)
                        issue_g(1 - b)

                    @pl.when(ci >= 2)
                    def _wsc():
                        wait_sc(b)

                    accumulate(b, ee_l, dc_l)

                    @pl.when(ci + 2 < nch)
                    def _pf():
                        issue_sidl(b, cbase(ci + 2))

            def chunk_pair(i2, ccarry):
                step(0, 2 * i2)
                step(1, 2 * i2 + 1)
                return ccarry

            lax.fori_loop(0, (nch + 1) // 2, chunk_pair, 0)

            @pl.when(nch > 0)
            def _dr0():
                wait_sc(0)

            @pl.when(nch > 1)
            def _dr1():
                wait_sc(1)

            pltpu.sync_copy(sacc_v.at[pl.ds(srow0, NS)], acc_v)

            for g in range(NS // 16):
                den16 = dac_v[pl.ds(g * 16, 16)]
                rec16 = 1.0 / (den16 + EPS)
                for r_ in range(16):
                    rr = g * 16 + r_
                    rec = _vsplat(rec16, r_)
                    for k in range(FC):
                        acc_v[rr, pl.ds(k * 16, 16)] = (
                            rec * acc_v[rr, pl.ds(k * 16, 16)])

            pltpu.sync_copy(acc_v, num_hbm.at[pl.ds(m0, NS)])
            return carry

        lax.fori_loop(0, nslices, slice_body, 0)

    return sc_gat


_sc_gat_128 = _make_sc_gat(8)


def _st_out(h, asv, adv, s_ref, t_ref):
    s_ref[...] = jnp.sum(h * asv, axis=1).reshape(1, 4, 128)
    t_ref[...] = jnp.sum(h * adv, axis=1).reshape(1, 4, 128)


_ST_SPEC = pl.BlockSpec((1, 4, 128), lambda i: (i, 0, 0))
_ST_SHAPE = jax.ShapeDtypeStruct((N_PAD // R, 4, 128), _f32)


def _tc_first(x, wt, asv, adv):
    din = x.shape[1]

    def body(x_ref, w_ref, as_ref, ad_ref, h_ref, s_ref, t_ref):
        xb = x_ref[...]
        xin = xb / jnp.maximum(jnp.sum(jnp.abs(xb), axis=1, keepdims=True),
                               1e-12)
        h = jnp.dot(xin, w_ref[...], preferred_element_type=_f32)
        h_ref[...] = h
        _st_out(h, as_ref[...], ad_ref[...], s_ref, t_ref)

    return pl.pallas_call(
        body,
        grid=(N_PAD // R,),
        in_specs=[
            pl.BlockSpec((R, din), lambda i: (i, 0)),
            pl.BlockSpec((din, 128), lambda i: (0, 0)),
            pl.BlockSpec((1, 128), lambda i: (0, 0)),
            pl.BlockSpec((1, 128), lambda i: (0, 0)),
        ],
        out_specs=[
            pl.BlockSpec((R, 128), lambda i: (i, 0)),
            _ST_SPEC,
            _ST_SPEC,
        ],
        out_shape=[
            jax.ShapeDtypeStruct((N_PAD, 128), _f32),
            _ST_SHAPE,
            _ST_SHAPE,
        ],
    )(x, wt, asv, adv)


def _tc_mid(num, bprev, wt, asv, adv, dout):
    def body(n_ref, b_ref, w_ref, as_ref, ad_ref, h_ref, s_ref, t_ref):
        xin = jnp.tanh(n_ref[...] + b_ref[...])
        h = jnp.dot(xin, w_ref[...], preferred_element_type=_f32)
        h_ref[...] = h
        _st_out(h, as_ref[...], ad_ref[...], s_ref, t_ref)

    return pl.pallas_call(
        body,
        grid=(N_PAD // R,),
        in_specs=[
            pl.BlockSpec((R, 128), lambda i: (i, 0)),
            pl.BlockSpec((1, 128), lambda i: (0, 0)),
            pl.BlockSpec((128, dout), lambda i: (0, 0)),
            pl.BlockSpec((1, dout), lambda i: (0, 0)),
            pl.BlockSpec((1, dout), lambda i: (0, 0)),
        ],
        out_specs=[
            pl.BlockSpec((R, dout), lambda i: (i, 0)),
            _ST_SPEC,
            _ST_SPEC,
        ],
        out_shape=[
            jax.ShapeDtypeStruct((N_PAD, dout), _f32),
            _ST_SHAPE,
            _ST_SHAPE,
        ],
    )(num, bprev, wt, asv, adv)


def _tc_mask(num4, b4, wm1t, bm1, wm2t, bm2, wm3t, bm3, wm4d, bm4ds,
             wt5, asv, adv):
    def body(n_ref, b_ref, w1_ref, b1_ref, w2_ref, b2_ref, w3_ref, b3_ref,
             w4_ref, b4d_ref, w5_ref, as_ref, ad_ref, h_ref, s_ref, t_ref):
        latent = jnp.tanh(n_ref[...] + b_ref[...])
        m = jnp.tanh(jnp.dot(latent, w1_ref[...],
                             preferred_element_type=_f32) + b1_ref[...])
        m = jnp.tanh(jnp.dot(m, w2_ref[...],
                             preferred_element_type=_f32) + b2_ref[...])
        m = jnp.tanh(jnp.dot(m, w3_ref[...],
                             preferred_element_type=_f32) + b3_ref[...])
        logit = jnp.sum(m * w4_ref[...] + b4d_ref[...], axis=1,
                        keepdims=True)
        sa = jax.nn.sigmoid(logit)
        d0 = latent * sa
        h = jnp.dot(d0, w5_ref[...], preferred_element_type=_f32)
        h_ref[...] = h
        _st_out(h, as_ref[...], ad_ref[...], s_ref, t_ref)

    return pl.pallas_call(
        body,
        grid=(N_PAD // R,),
        in_specs=[
            pl.BlockSpec((R, 128), lambda i: (i, 0)),
            pl.BlockSpec((1, 128), lambda i: (0, 0)),
            pl.BlockSpec((128, 64), lambda i: (0, 0)),
            pl.BlockSpec((1, 64), lambda i: (0, 0)),
            pl.BlockSpec((64, 16), lambda i: (0, 0)),
            pl.BlockSpec((1, 16), lambda i: (0, 0)),
            pl.BlockSpec((16, 16), lambda i: (0, 0)),
            pl.BlockSpec((1, 16), lambda i: (0, 0)),
            pl.BlockSpec((1, 16), lambda i: (0, 0)),
            pl.BlockSpec((1, 16), lambda i: (0, 0)),
            pl.BlockSpec((128, 128), lambda i: (0, 0)),
            pl.BlockSpec((1, 128), lambda i: (0, 0)),
            pl.BlockSpec((1, 128), lambda i: (0, 0)),
        ],
        out_specs=[
            pl.BlockSpec((R, 128), lambda i: (i, 0)),
            _ST_SPEC,
            _ST_SPEC,
        ],
        out_shape=[
            jax.ShapeDtypeStruct((N_PAD, 128), _f32),
            _ST_SHAPE,
            _ST_SHAPE,
        ],
    )(num4, b4, wm1t, bm1, wm2t, bm2, wm3t, bm3, wm4d, bm4ds, wt5, asv, adv)


def _tc_final(num8, b8):
    def body(n_ref, b_ref, o_ref):
        o_ref[...] = jnp.tanh(n_ref[...] + b_ref[...])

    return pl.pallas_call(
        body,
        grid=(N_PAD // R,),
        in_specs=[
            pl.BlockSpec((R, 128), lambda i: (i, 0)),
            pl.BlockSpec((1, 128), lambda i: (0, 0)),
        ],
        out_specs=pl.BlockSpec((R, 128), lambda i: (i, 0)),
        out_shape=jax.ShapeDtypeStruct((N_PAD, 128), _f32),
    )(num8, b8)


def kernel(x, edge_index, batch, epoch, params):
    src32 = edge_index[0].astype(_i32)
    dst32 = edge_index[1].astype(_i32)
    dsts_s, srcs_s = lax.sort((dst32, src32), num_keys=1)
    pad_d = jnp.full((E_PAD - E_RAW,), N_RAW + 100, _i32)
    pad_s = jnp.zeros((E_PAD - E_RAW,), _i32)
    dsts_p = jnp.concatenate([dsts_s, pad_d])
    srcs_p = jnp.concatenate([srcs_s, pad_s])
    rowptr = jnp.searchsorted(dsts_p, jnp.arange(RP_LEN, dtype=_i32),
                              side="left").astype(_i32)
    cuts = dsts_s[(jnp.arange(1, 32) * E_RAW) // 32]
    nbmid = (cuts // 64) * 64
    nb = jnp.concatenate([
        jnp.zeros((1,), _i32), nbmid.astype(_i32),
        jnp.full((NB_LEN - 32,), N_PAD, _i32)])

    def row(v, w=128):
        out = jnp.zeros((1, w), _f32)
        return out.at[0, : v.shape[0]].set(v)

    wts = {}
    for i in range(1, 9):
        wts[i] = params["Wc%d" % i].T
    w1t = jnp.zeros((48, 128), _f32).at[:42].set(wts[1])
    w8t = jnp.zeros((128, 128), _f32).at[:, :42].set(wts[8])
    asv = {i: row(params["asrc%d" % i]) for i in range(1, 9)}
    adv = {i: row(params["adst%d" % i]) for i in range(1, 9)}
    bs = {i: row(params["bc%d" % i]) for i in range(1, 9)}
    wm1t, wm2t, wm3t = (params["Wm1"].T, params["Wm2"].T, params["Wm3"].T)
    bm1, bm2, bm3 = (row(params["bm1"], 64), row(params["bm2"], 16),
                     row(params["bm3"], 16))
    wm4d = (params["Wm4"][1] - params["Wm4"][0]).reshape(1, 16)
    bm4d = params["bm4"][1] - params["bm4"][0]
    bm4ds = jnp.full((1, 16), bm4d / 16.0, _f32)

    xp = jnp.zeros((N_PAD, 48), _f32).at[:N_RAW, :42].set(x)

    zer = jnp.zeros((NS, 128), _f32)

    def agg(h, s, t):
        return _sc_gat_128(h, s.reshape(-1), t.reshape(-1), srcs_p, dsts_p,
                           rowptr, nb, zer)

    h, s, t = _tc_first(xp, w1t, asv[1], adv[1])
    num = agg(h, s, t)
    for i in (2, 3, 4):
        h, s, t = _tc_mid(num, bs[i - 1], wts[i], asv[i], adv[i], 128)
        num = agg(h, s, t)
    h, s, t = _tc_mask(num, bs[4], wm1t, bm1, wm2t, bm2, wm3t, bm3,
                       wm4d, bm4ds, wts[5], asv[5], adv[5])
    num = agg(h, s, t)
    for i in (6, 7):
        h, s, t = _tc_mid(num, bs[i - 1], wts[i], asv[i], adv[i], 128)
        num = agg(h, s, t)
    h, s, t = _tc_mid(num, bs[7], w8t, asv[8], adv[8], 128)
    num = agg(h, s, t)
    out = _tc_final(num, bs[8])
    return out[:N_RAW, :42]
